# baseline (speedup 1.0000x reference)
import functools

import jax
import jax.numpy as jnp
from jax import lax
from jax.experimental import pallas as pl
from jax.experimental.pallas import tpu as pltpu
from jax.experimental.pallas import tpu_sc as plsc

NUM_SPK = 100000
EMBD_DIM = 64
BATCH = 16384

NUM_CORES = 2
NUM_SUBCORES = 16
NW = NUM_CORES * NUM_SUBCORES
B_PER_W = BATCH // NW
IDX_CHUNK = 128
N_CHUNKS = B_PER_W // IDX_CHUNK
N_VEC = B_PER_W // 16

_mesh = plsc.VectorSubcoreMesh(core_axis_name="c", subcore_axis_name="s")


@functools.partial(
    pl.kernel,
    mesh=_mesh,
    compiler_params=pltpu.CompilerParams(
        use_tc_tiling_on_sc=True, needs_layout_passes=False
    ),
    out_type=jax.ShapeDtypeStruct((EMBD_DIM, BATCH), jnp.float32),
    scratch_types=[
        pltpu.VMEM((N_CHUNKS, IDX_CHUNK), jnp.int32),
        pltpu.VMEM((N_CHUNKS, IDX_CHUNK), jnp.int32),
        pltpu.VMEM((N_CHUNKS, IDX_CHUNK), jnp.int32),
        pltpu.VMEM((B_PER_W, 128), jnp.float32),
        pltpu.VMEM((EMBD_DIM, B_PER_W), jnp.float32),
        pltpu.SemaphoreType.DMA,
    ],
)
def _sc_gather(t128_hbm, idx_hbm, out_hbm, idx_v, row_v, off_v, rows_v,
               tb_v, sem):
    wid = lax.axis_index("s") * NUM_CORES + lax.axis_index("c")
    base = wid * B_PER_W
    pltpu.sync_copy(idx_hbm.at[pl.ds(wid * N_CHUNKS, N_CHUNKS)], idx_v)
    for t in range(N_VEC):
        c, s = t // 8, (t % 8) * 16
        i = idx_v[c, pl.ds(s, 16)]
        row_v[c, pl.ds(s, 16)] = lax.shift_right_logical(i, 1)
        off_v[c, pl.ds(s, 16)] = lax.shift_left(lax.bitwise_and(i, 1), 6)
    copies = []
    for j in range(N_CHUNKS):
        copies.append(
            pltpu.async_copy(
                t128_hbm.at[row_v.at[j]],
                rows_v.at[pl.ds(j * IDX_CHUNK, IDX_CHUNK)],
                sem,
            )
        )
    for c in copies:
        c.wait()

    lanes = lax.iota(jnp.int32, 16)

    def xpose(j, _):
        jvec = lax.broadcast(j, (16,))
        for k in range(N_VEC):
            c, s = k // 8, (k % 8) * 16
            bvec = lax.add(lanes, jnp.int32(k * 16))
            col = lax.add(off_v[c, pl.ds(s, 16)], jvec)
            val = plsc.load_gather(rows_v, [bvec, col])
            tb_v[j, pl.ds(k * 16, 16)] = val
        return 0

    lax.fori_loop(0, EMBD_DIM, xpose, 0)
    pltpu.sync_copy(tb_v, out_hbm.at[:, pl.ds(base, B_PER_W)])


def kernel(spk_inds, embedding_table):
    t128 = embedding_table.reshape(NUM_SPK // 2, 2 * EMBD_DIM)
    idx2d = spk_inds.astype(jnp.int32).reshape(NW * N_CHUNKS, IDX_CHUNK)
    out_t = _sc_gather(t128, idx2d)
    return out_t.T


# speedup vs baseline: 1.5254x; 1.5254x over previous
import functools

import jax
import jax.numpy as jnp
from jax import lax
from jax.experimental import pallas as pl
from jax.experimental.pallas import tpu as pltpu
from jax.experimental.pallas import tpu_sc as plsc

NUM_SPK = 100000
EMBD_DIM = 64
BATCH = 16384

NUM_CORES = 2
NUM_SUBCORES = 16
NW = NUM_CORES * NUM_SUBCORES
B_PER_W = BATCH // NW
IDX_CHUNK = 128
N_CHUNKS = B_PER_W // IDX_CHUNK

_mesh = plsc.VectorSubcoreMesh(core_axis_name="c", subcore_axis_name="s")


@functools.partial(
    pl.kernel,
    mesh=_mesh,
    compiler_params=pltpu.CompilerParams(
        use_tc_tiling_on_sc=True, needs_layout_passes=False
    ),
    out_type=jax.ShapeDtypeStruct((EMBD_DIM, BATCH), jnp.float32),
    scratch_types=[
        pltpu.VMEM((B_PER_W + 16,), jnp.int32),
        pltpu.VMEM((B_PER_W, EMBD_DIM), jnp.float32),
        pltpu.VMEM((EMBD_DIM, B_PER_W), jnp.float32),
        pltpu.SemaphoreType.DMA,
        pltpu.SemaphoreType.DMA,
    ],
)
def _sc_gather(table_hbm, idx_hbm, out_hbm, idx_v, rows_v, tb_v, sem, sem2):
    wid = lax.axis_index("s") * NUM_CORES + lax.axis_index("c")
    base = wid * B_PER_W
    pltpu.sync_copy(idx_hbm.at[pl.ds(base, B_PER_W)], idx_v.at[pl.ds(0, B_PER_W)])

    half = B_PER_W // 2

    def issue_half(h, s):
        def issue(k, _):
            b0 = h * half + k * 16
            v = idx_v[pl.ds(b0, 16)]
            for i in range(16):
                pltpu.async_copy(
                    table_hbm.at[pl.ds(v[i], 1), :],
                    rows_v.at[pl.ds(b0 + i, 1), :],
                    s,
                )
            return 0

        lax.fori_loop(0, half // 16, issue, 0)

    lanes = lax.iota(jnp.int32, 16)

    def xpose_half(h):
        def xpose(j, _):
            col = lax.broadcast(j, (16,))
            for k in range(half // 16):
                bvec = lax.add(lanes, jnp.int32(h * half + k * 16))
                val = plsc.load_gather(rows_v, [bvec, col])
                tb_v[j, pl.ds(h * half + k * 16, 16)] = val
            return 0

        lax.fori_loop(0, EMBD_DIM, xpose, 0)

    issue_half(0, sem)
    issue_half(1, sem2)
    pltpu.make_async_copy(
        table_hbm.at[pl.ds(0, half), :], rows_v.at[pl.ds(0, half), :], sem
    ).wait()
    xpose_half(0)
    pltpu.make_async_copy(
        table_hbm.at[pl.ds(0, half), :], rows_v.at[pl.ds(half, half), :], sem2
    ).wait()
    xpose_half(1)
    pltpu.sync_copy(tb_v, out_hbm.at[:, pl.ds(base, B_PER_W)])


def kernel(spk_inds, embedding_table):
    out_t = _sc_gather(embedding_table, spk_inds.astype(jnp.int32))
    return out_t.T
